# baseline (speedup 1.0000x reference)
import functools

import jax
import jax.numpy as jnp
from jax import lax
from jax.experimental import pallas as pl
from jax.experimental.pallas import tpu as pltpu
from jax.experimental.pallas import tpu_sc as plsc

_NUM_CORES = 2
_NUM_SUBCORES = 16
_NUM_WORKERS = _NUM_CORES * _NUM_SUBCORES


@functools.lru_cache(maxsize=None)
def _make_bcast_copy3(batch: int, seq: int, h_dim: int, sizes: tuple, nbuf: int):
    rows_per_w = seq // _NUM_WORKERS
    assert seq % _NUM_WORKERS == 0 and sum(sizes) == rows_per_w
    n_chunks = len(sizes)
    assert n_chunks >= 2 * nbuf >= 4
    bufrows = [
        max(sizes[c] for c in range(n_chunks) if c % nbuf == i) for i in range(nbuf)
    ]
    offs = [sum(sizes[:i]) for i in range(n_chunks)]
    assert sum(bufrows) * h_dim <= 131071

    mesh = plsc.VectorSubcoreMesh(core_axis_name="c", subcore_axis_name="s")

    @functools.partial(
        pl.kernel,
        mesh=mesh,
        out_type=jax.ShapeDtypeStruct((batch, seq, h_dim), jnp.float32),
        scratch_types=[
            pltpu.VMEM_SHARED(
                (_NUM_SUBCORES, nbuf, max(bufrows), h_dim), jnp.float32
            ),
            [pltpu.VMEM((bufrows[i], h_dim), jnp.float32) for i in range(nbuf)],
            [pltpu.SemaphoreType.DMA for _ in range(nbuf)],
            [pltpu.SemaphoreType.DMA for _ in range(nbuf)],
            [pltpu.SemaphoreType.DMA for _ in range(nbuf)],
        ],
    )
    def bcast_copy(table, out, spmem, bufs, ssems, xsems, wsems):
        sid = lax.axis_index("s")
        wid = sid * _NUM_CORES + lax.axis_index("c")
        base = wid * rows_per_w

        def spslot(c):
            slot = spmem.at[sid, c % nbuf]
            return slot.at[pl.ds(0, sizes[c])]

        def tbuf(c):
            buf = bufs[c % nbuf]
            if sizes[c] == bufrows[c % nbuf]:
                return buf
            return buf.at[pl.ds(0, sizes[c])]

        def spread(c):
            return pltpu.make_async_copy(
                table.at[pl.ds(base + offs[c], sizes[c])], spslot(c), ssems[c % nbuf]
            )

        spreads = [spread(c) for c in range(n_chunks)]
        writes = [None] * n_chunks
        for c in range(nbuf):
            spreads[c].start()
        for c in range(n_chunks):
            spreads[c].wait()
            if c >= nbuf:
                for w in writes[c - nbuf]:
                    w.wait()
            xf = pltpu.make_async_copy(spslot(c), tbuf(c), xsems[c % nbuf])
            xf.start()
            xf.wait()
            if c + nbuf < n_chunks:
                spreads[c + nbuf].start()
            wr = [
                pltpu.make_async_copy(
                    tbuf(c),
                    out.at[b, pl.ds(base + offs[c], sizes[c])],
                    wsems[c % nbuf],
                )
                for b in range(batch)
            ]
            for w in wr:
                w.start()
            writes[c] = wr
        for j in range(n_chunks - nbuf, n_chunks):
            for w in writes[j]:
                w.wait()

    return bcast_copy


def kernel(x, pos_embedding):
    batch, seq = x.shape
    _, h_dim = pos_embedding.shape
    table = pos_embedding[:seq] if pos_embedding.shape[0] != seq else pos_embedding
    return _make_bcast_copy3(batch, seq, h_dim, (32, 32, 32, 32, 32, 32, 32, 32), 2)(table)


# speedup vs baseline: 1.0447x; 1.0447x over previous
import functools

import jax
import jax.numpy as jnp
from jax import lax
from jax.experimental import pallas as pl
from jax.experimental.pallas import tpu as pltpu
from jax.experimental.pallas import tpu_sc as plsc

_NUM_CORES = 2
_NUM_SUBCORES = 16
_NUM_WORKERS = _NUM_CORES * _NUM_SUBCORES


@functools.lru_cache(maxsize=None)
def _make_bcast_copy(batch: int, seq: int, h_dim: int, sizes: tuple, nbuf: int):
    rows_per_w = seq // _NUM_WORKERS
    assert seq % _NUM_WORKERS == 0 and sum(sizes) == rows_per_w
    n_chunks = len(sizes)
    assert n_chunks >= nbuf >= 2
    bufrows = [
        max(sizes[c] for c in range(n_chunks) if c % nbuf == i) for i in range(nbuf)
    ]
    offs = [sum(sizes[:i]) for i in range(n_chunks)]
    assert sum(bufrows) * h_dim <= 131071

    mesh = plsc.VectorSubcoreMesh(core_axis_name="c", subcore_axis_name="s")

    @functools.partial(
        pl.kernel,
        mesh=mesh,
        out_type=jax.ShapeDtypeStruct((batch, seq, h_dim), jnp.float32),
        scratch_types=[
            [pltpu.VMEM((bufrows[i], h_dim), jnp.float32) for i in range(nbuf)],
            [pltpu.SemaphoreType.DMA for _ in range(nbuf)],
            [pltpu.SemaphoreType.DMA for _ in range(nbuf)],
        ],
    )
    def bcast_copy(table, out, bufs, rsems, wsems):
        wid = lax.axis_index("s") * _NUM_CORES + lax.axis_index("c")
        base = wid * rows_per_w

        def src(c):
            buf = bufs[c % nbuf]
            if sizes[c] == bufrows[c % nbuf]:
                return buf
            return buf.at[pl.ds(0, sizes[c])]

        def read(c):
            return pltpu.make_async_copy(
                table.at[pl.ds(base + offs[c], sizes[c])],
                src(c),
                rsems[c % nbuf],
            )

        reads = [read(c) for c in range(n_chunks)]
        writes = [None] * n_chunks
        reads[0].start()
        for c in range(n_chunks):
            reads[c].wait()
            wr = [
                pltpu.make_async_copy(
                    src(c),
                    out.at[b, pl.ds(base + offs[c], sizes[c])],
                    wsems[c % nbuf],
                )
                for b in range(batch)
            ]
            for w in wr:
                w.start()
            writes[c] = wr
            if c + 1 < n_chunks:
                j = c + 1 - nbuf
                if j >= 0:
                    for w in writes[j]:
                        w.wait()
                reads[c + 1].start()
        for j in range(max(0, n_chunks - nbuf), n_chunks):
            for w in writes[j]:
                w.wait()

    return bcast_copy


def kernel(x, pos_embedding):
    batch, seq = x.shape
    _, h_dim = pos_embedding.shape
    table = pos_embedding[:seq] if pos_embedding.shape[0] != seq else pos_embedding
    return _make_bcast_copy(batch, seq, h_dim, (32, 56, 56, 56, 56), 2)(table)
